# baseline (speedup 1.0000x reference)
import functools

import jax
import jax.numpy as jnp
from jax import lax
from jax.experimental import pallas as pl
from jax.experimental.pallas import tpu as pltpu
from jax.experimental.pallas import tpu_sc as plsc
from jax._src.pallas import mpmd as _pl_mpmd

_IBIG = 1 << 20
_NC, _NS, _L = 2, 16, 16
_NW = _NC * _NS
_RPC = 2
_BB = 64


def _sc_select(nop_ref, pt_ref, tbu_ref, out_ref, ptb, nopb, tbub, posb,
               *, rows, n_jobs, n_mas, n_trs, n_ops, n_cols):
    cid = lax.axis_index("c")
    sid = lax.axis_index("s")
    wid = sid * _NC + cid
    base = wid * rows
    n_jc = nopb.shape[1] // _L
    lane = lax.iota(jnp.int32, _L)

    def chunk_body(c, carry):
        r0 = base + c * _RPC
        pltpu.sync_copy(pt_ref.at[pl.ds(r0, _RPC)], ptb)
        pltpu.sync_copy(nop_ref.at[pl.ds(r0, _RPC)], nopb)
        pltpu.sync_copy(tbu_ref.at[pl.ds(r0, _RPC)], tbub)
        for r in range(_RPC):
            curval = jnp.full((_L,), jnp.inf, jnp.float32)
            curkey = jnp.full((_L,), _IBIG, jnp.int32)
            rvec = jnp.full((_L,), r, jnp.int32)
            for jc in range(n_jc):
                idx16 = nopb[r, pl.ds(jc * _L, _L)]
                jkey = (jc * _L + lane) * n_mas
                pad = n_jobs - jc * _L
                for m in range(n_mas):
                    v = plsc.load_gather(ptb, [rvec, idx16 + m * n_ops])
                    better = (v < curval) & (v != 0.0)
                    if pad < _L:
                        better = better & (lane < pad)
                    curval = jnp.where(better, v, curval)
                    curkey = jnp.where(better, jkey + m, curkey)
            minv = jnp.min(curval)
            fkey = jnp.min(jnp.where(curval == minv, curkey, _IBIG))
            fkey = jnp.where(minv == jnp.inf, 0, fkey)
            tv = tbub[r]
            tkey = jnp.min(jnp.where(tv == jnp.min(tv), lane, _L))
            pos = (r0 + r) * n_cols + 1 + fkey * n_trs + tkey
            posb[r] = jnp.full((_L,), pos, jnp.int32)
        pltpu.sync_copy(posb, out_ref.at[pl.ds(r0, _RPC)])
        return carry

    lax.fori_loop(0, rows // _RPC, chunk_body, 0)


def _sc_scatter(zeros_ref, pos_ref, out_ref, posb, onesb, *, rows):
    del zeros_ref
    cid = lax.axis_index("c")
    sid = lax.axis_index("s")
    wid = sid * _NC + cid
    base = wid * rows
    lane = lax.iota(jnp.int32, _L)
    zero16 = jnp.zeros((_L,), jnp.int32)
    onesb[...] = jnp.ones((_L,), jnp.float32)
    pltpu.sync_copy(pos_ref.at[pl.ds(base, rows)], posb)
    for g in range(rows // _L):
        idx16 = plsc.load_gather(posb, [g * _L + lane, zero16])
        pltpu.sync_copy(onesb, out_ref.at[idx16])


def _tc_fill(out_ref):
    out_ref[...] = jnp.zeros(out_ref.shape, jnp.float32)


def kernel(job_done, machine_busy_until, truck_location, next_op, proc_times,
           truck_busy_until, action_mask):
    B, n_jobs = job_done.shape
    n_mas = machine_busy_until.shape[1]
    n_trs = truck_location.shape[1]
    n_ops = proc_times.shape[2]
    n_cols = 1 + n_jobs * n_mas * n_trs
    rows = B // _NW

    jpad = (-n_jobs) % _L
    nop_p = jnp.pad(next_op, ((0, 0), (0, jpad)))
    tbu_p = jnp.pad(truck_busy_until, ((0, 0), (0, _L - n_trs)),
                    constant_values=jnp.inf)
    pt2 = proc_times.reshape(B, n_mas * n_ops)

    sel = functools.partial(_sc_select, rows=rows, n_jobs=n_jobs,
                            n_mas=n_mas, n_trs=n_trs, n_ops=n_ops,
                            n_cols=n_cols)
    pos16 = pl.kernel(
        sel,
        out_type=jax.ShapeDtypeStruct((B, _L), jnp.int32),
        mesh=plsc.VectorSubcoreMesh(core_axis_name="c", subcore_axis_name="s",
                                    num_cores=_NC, num_subcores=_NS),
        compiler_params=pltpu.CompilerParams(needs_layout_passes=False),
        scratch_types=[
            pltpu.VMEM((_RPC, n_mas * n_ops), jnp.float32),
            pltpu.VMEM((_RPC, n_jobs + jpad), jnp.int32),
            pltpu.VMEM((_RPC, _L), jnp.float32),
            pltpu.VMEM((_RPC, _L), jnp.int32),
        ],
    )(nop_p, pt2, tbu_p)

    zeros = pl.pallas_call(
        _tc_fill,
        grid=(B // _BB,),
        out_specs=pl.BlockSpec((_BB, n_cols), lambda i: (i, 0)),
        out_shape=jax.ShapeDtypeStruct((B, n_cols), jnp.float32),
    )()

    scat = functools.partial(_sc_scatter, rows=rows)
    mesh = plsc.VectorSubcoreMesh(core_axis_name="c", subcore_axis_name="s",
                                  num_cores=_NC, num_subcores=_NS)
    logits_flat = _pl_mpmd._mpmd_map(
        [(mesh, scat)],
        out_types=jax.ShapeDtypeStruct((B * n_cols,), jnp.float32),
        input_output_aliases={0: 0},
        compiler_params=pltpu.CompilerParams(needs_layout_passes=False),
        scratch_types=[
            pltpu.VMEM((rows, _L), jnp.int32),
            pltpu.VMEM((_L,), jnp.float32),
        ],
    )(zeros.reshape(B * n_cols), pos16)
    logits = logits_flat.reshape(B, n_cols)
    return (logits, action_mask)


# speedup vs baseline: 10.0045x; 10.0045x over previous
import functools

import jax
import jax.numpy as jnp
from jax import lax
from jax.experimental import pallas as pl
from jax.experimental.pallas import tpu as pltpu
from jax.experimental.pallas import tpu_sc as plsc

_IBIG = 1 << 20
_NC, _NS, _L = 2, 16, 16
_NW = _NC * _NS
_RPC = 8
_NACC = 4
_BB = 64


def _sc_select(nop_ref, pt_ref, tbu_ref, out_ref, ptb, nopb, tbub, actb, sem,
               *, rows, n_jobs, n_mas, n_trs, n_ops):
    cid = lax.axis_index("c")
    sid = lax.axis_index("s")
    wid = sid * _NC + cid
    base = wid * rows
    n_jc = nopb.shape[1] // _L
    nchunk = rows // _RPC
    lane = lax.iota(jnp.int32, _L)

    pltpu.sync_copy(nop_ref.at[pl.ds(base, rows)], nopb)
    pltpu.sync_copy(tbu_ref.at[pl.ds(base, rows)], tbub)
    pltpu.async_copy(pt_ref.at[pl.ds(base, _RPC)], ptb.at[0], sem)
    pltpu.async_copy(pt_ref.at[pl.ds(base + _RPC, _RPC)], ptb.at[1], sem)

    def chunk_body(c, carry):
        buf = lax.rem(c, 2)
        pltpu.make_async_copy(pt_ref.at[pl.ds(base, _RPC)], ptb.at[0], sem).wait()
        bvec = jnp.full((_L,), buf, jnp.int32)

        def row_body(r, carry2):
            row = c * _RPC + r
            rvec = jnp.full((_L,), r, jnp.int32)
            vals = [jnp.full((_L,), jnp.inf, jnp.float32) for _ in range(_NACC)]
            keys = [jnp.full((_L,), _IBIG, jnp.int32) for _ in range(_NACC)]
            for jc in range(n_jc):
                idx16 = nopb[row, pl.ds(jc * _L, _L)]
                jkey = (jc * _L + lane) * n_mas
                pad = n_jobs - jc * _L
                for m in range(n_mas):
                    a = m % _NACC
                    v = plsc.load_gather(ptb, [bvec, rvec, idx16 + m * n_ops])
                    better = (v < vals[a]) & (v != 0.0)
                    if pad < _L:
                        better = better & (lane < pad)
                    vals[a] = jnp.where(better, v, vals[a])
                    keys[a] = jnp.where(better, jkey + m, keys[a])
            vm, km = vals[0], keys[0]
            for a in range(1, _NACC):
                take = (vals[a] < vm) | ((vals[a] == vm) & (keys[a] < km))
                vm = jnp.where(take, vals[a], vm)
                km = jnp.where(take, keys[a], km)
            minv = jnp.min(vm)
            fkey = jnp.min(jnp.where(vm == minv, km, _IBIG))
            fkey = jnp.where(minv == jnp.inf, 0, fkey)
            tv = tbub[row]
            tkey = jnp.min(jnp.where(tv == jnp.min(tv), lane, _L))
            act = 1 + fkey * n_trs + tkey
            actb[r] = jnp.full((_L,), act, jnp.int32)
            return carry2

        lax.fori_loop(0, _RPC, row_body, 0)
        pltpu.sync_copy(actb, out_ref.at[pl.ds(base + c * _RPC, _RPC)])

        @pl.when(c + 2 < nchunk)
        def _prefetch():
            pltpu.async_copy(pt_ref.at[pl.ds(base + (c + 2) * _RPC, _RPC)],
                             ptb.at[buf], sem)

        return carry

    lax.fori_loop(0, nchunk, chunk_body, 0)


def _tc_onehot(act_ref, out_ref):
    act = act_ref[:, :1]
    n_cols = out_ref.shape[1]
    col = lax.broadcasted_iota(jnp.int32, (act_ref.shape[0], n_cols), 1)
    out_ref[...] = jnp.where(col == act, 1.0, 0.0).astype(jnp.float32)


def kernel(job_done, machine_busy_until, truck_location, next_op, proc_times,
           truck_busy_until, action_mask):
    B, n_jobs = job_done.shape
    n_mas = machine_busy_until.shape[1]
    n_trs = truck_location.shape[1]
    n_ops = proc_times.shape[2]
    n_cols = 1 + n_jobs * n_mas * n_trs
    rows = B // _NW

    jpad = (-n_jobs) % _L
    nop_p = jnp.pad(next_op, ((0, 0), (0, jpad)))
    tbu_p = jnp.pad(truck_busy_until, ((0, 0), (0, _L - n_trs)),
                    constant_values=jnp.inf)
    pt2 = proc_times.reshape(B, n_mas * n_ops)

    sel = functools.partial(_sc_select, rows=rows, n_jobs=n_jobs,
                            n_mas=n_mas, n_trs=n_trs, n_ops=n_ops)
    act16 = pl.kernel(
        sel,
        out_type=jax.ShapeDtypeStruct((B, _L), jnp.int32),
        mesh=plsc.VectorSubcoreMesh(core_axis_name="c", subcore_axis_name="s",
                                    num_cores=_NC, num_subcores=_NS),
        compiler_params=pltpu.CompilerParams(needs_layout_passes=False),
        scratch_types=[
            pltpu.VMEM((2, _RPC, n_mas * n_ops), jnp.float32),
            pltpu.VMEM((rows, n_jobs + jpad), jnp.int32),
            pltpu.VMEM((rows, _L), jnp.float32),
            pltpu.VMEM((_RPC, _L), jnp.int32),
            pltpu.SemaphoreType.DMA,
        ],
    )(nop_p, pt2, tbu_p)

    logits = pl.pallas_call(
        _tc_onehot,
        grid=(B // _BB,),
        in_specs=[pl.BlockSpec((_BB, _L), lambda i: (i, 0))],
        out_specs=pl.BlockSpec((_BB, n_cols), lambda i: (i, 0)),
        out_shape=jax.ShapeDtypeStruct((B, n_cols), jnp.float32),
    )(act16)
    return (logits, action_mask)
